# Initial kernel scaffold; baseline (speedup 1.0000x reference)
import functools

import jax
import jax.numpy as jnp
from jax import lax
from jax.experimental import pallas as pl
from jax.experimental.pallas import tpu as pltpu
from jax.experimental.pallas import tpu_sc as plsc

GRID = 0.001
DEFAULT_MAIN_VIEW = 0

_SIGN = jnp.int32(-2**31)
_C1 = 1
_C0 = 34464


def _ult(a, b):
    return (a ^ _SIGN) < (b ^ _SIGN)


def _uge_c(a, c):
    return ((a ^ _SIGN) >= (jnp.int32(c) ^ _SIGN)).astype(jnp.int32)


def _add64(ah, al, bh, bl):
    lo = al + bl
    carry = _ult(lo, al).astype(jnp.int32)
    return ah + bh + carry, lo


def _sub64(ah, al, bh, bl):
    borrow = _ult(al, bl).astype(jnp.int32)
    return ah - bh - borrow, al - bl


def _mul64_100000(h, l):
    l1 = lax.shift_right_logical(l, 16)
    l0 = l & jnp.int32(0xFFFF)
    plo = l0 * jnp.int32(_C0)
    mid = l0 * jnp.int32(_C1) + l1 * jnp.int32(_C0)
    lo = plo + lax.shift_left(mid & jnp.int32(0xFFFF), 16)
    carry = _ult(lo, plo).astype(jnp.int32)
    hi = lax.shift_right_logical(mid, 16) + l1 * jnp.int32(_C1) + carry \
        + h * jnp.int32(100000)
    return hi, lo


def _split64(qf):
    neg = qf < 0.0
    a = jnp.abs(qf)
    hif = jnp.floor(a * jnp.float32(2.0**-32))
    ha = hif.astype(jnp.int32)
    remf = a - hif * jnp.float32(2.0**32)
    big = remf >= jnp.float32(2.0**31)
    lo_small = jnp.where(big, jnp.float32(0.0), remf).astype(jnp.int32)
    lo_big = jnp.where(big, remf - jnp.float32(2.0**31),
                       jnp.float32(0.0)).astype(jnp.int32) | _SIGN
    la = jnp.where(big, lo_big, lo_small)
    nlo = -la
    nhi = -ha - jnp.where(la == jnp.int32(0), jnp.int32(0), jnp.int32(1))
    return jnp.where(neg, nhi, ha), jnp.where(neg, nlo, la)


def _masked_min64(hi, lo, mask):
    mh = jnp.min(jnp.where(mask, hi, jnp.int32(2**31 - 1)))
    sel = mask & (hi == mh)
    ml = jnp.min(jnp.where(sel, lo ^ _SIGN, jnp.int32(2**31 - 1))) ^ _SIGN
    return mh, ml


def _masked_max64(hi, lo, mask):
    mh = jnp.max(jnp.where(mask, hi, _SIGN))
    sel = mask & (hi == mh)
    ml = jnp.max(jnp.where(sel, lo ^ _SIGN, _SIGN)) ^ _SIGN
    return mh, ml


def _key_kernel(q0_ref, q1_ref, q2_ref, pad_ref, skh_ref, skl_ref, meta_ref):
    q0f = q0_ref[0]
    q1f = q1_ref[0]
    q2f = q2_ref[0]
    padm = pad_ref[0] != 0
    valid = ~padm
    any_valid = jnp.any(valid)

    q0 = q0f.astype(jnp.int32)
    q1h, q1l = _split64(q1f)
    q2h, q2l = _split64(q2f)

    m0 = jnp.min(jnp.where(valid, q0, jnp.int32(2**31 - 1)))
    m1h, m1l = _masked_min64(q1h, q1l, valid)
    m2h, m2l = _masked_min64(q2h, q2l, valid)
    zero = jnp.int32(0)
    m0 = jnp.where(any_valid, m0, zero)
    m1h = jnp.where(any_valid, m1h, zero)
    m1l = jnp.where(any_valid, m1l, zero)
    m2h = jnp.where(any_valid, m2h, zero)
    m2l = jnp.where(any_valid, m2l, zero)

    d0 = q0 - m0
    d1h, d1l = _sub64(q1h, q1l, m1h, m1l)
    d2h, d2l = _sub64(q2h, q2l, m2h, m2l)

    x1h, x1l = _masked_max64(d1h, d1l, valid)
    x2h, x2l = _masked_max64(d2h, d2l, valid)
    x1h = jnp.where(any_valid, x1h, zero)
    x1l = jnp.where(any_valid, x1l, zero)
    x2h = jnp.where(any_valid, x2h, zero)
    x2l = jnp.where(any_valid, x2l, zero)

    p0 = (d0 & 1) == 1
    p01 = ((d0 ^ d1l) & 1) == 1
    a1h, a1l = _sub64(x1h, x1l, d1h, d1l)
    s1h = jnp.where(p0, a1h, d1h)
    s1l = jnp.where(p0, a1l, d1l)
    a2h, a2l = _sub64(x2h, x2l, d2h, d2l)
    s2h = jnp.where(p01, a2h, d2h)
    s2l = jnp.where(p01, a2l, d2l)

    k1h, k1l = _add64(jnp.zeros_like(d0), d0 * jnp.int32(100000), s1h, s1l)
    k2h, k2l = _mul64_100000(k1h, k1l)
    kh, kl = _add64(k2h, k2l, s2h, s2l)

    vh, vl = _masked_max64(kh, kl, valid)
    ph, plo = _add64(vh, vl, zero, jnp.int32(1))
    kh = jnp.where(padm, ph, kh)
    kl = jnp.where(padm, plo, kl)

    ukh = kh ^ _SIGN
    mnh_b = jnp.min(kh)
    mnl = jnp.min(jnp.where(kh == mnh_b, kl ^ _SIGN, jnp.int32(2**31 - 1))) ^ _SIGN
    mxh_b = jnp.max(kh)
    mxl = jnp.max(jnp.where(kh == mxh_b, kl ^ _SIGN, _SIGN)) ^ _SIGN
    umnh = mnh_b ^ _SIGN
    umxh = mxh_b ^ _SIGN
    skh, skl = _sub64(ukh, kl, umnh, mnl)
    sph, spl = _sub64(ph ^ _SIGN, plo, umnh, mnl)
    rh, rl = _sub64(umxh, mxl, umnh, mnl)
    cnt_lo = jnp.int32(1) + _uge_c(rl, 1 << 8) + _uge_c(rl, 1 << 16) \
        + _uge_c(rl, 1 << 24)
    cnt_hi = jnp.int32(5) + _uge_c(rh, 1 << 8) + _uge_c(rh, 1 << 16) \
        + _uge_c(rh, 1 << 24)
    npass = jnp.where(rh == 0, cnt_lo, cnt_hi)

    skh_ref[0] = skh
    skl_ref[0] = skl
    meta_ref[0, 0:1, :] = jnp.full((1, 128), sph, jnp.int32)
    meta_ref[0, 1:2, :] = jnp.full((1, 128), spl, jnp.int32)
    meta_ref[0, 2:3, :] = jnp.full((1, 128), npass, jnp.int32)


def _compute_keys_tc(q0t, q1t, q2t, padt):
    B, N = q0t.shape
    r = lambda x: x.reshape(B, N // 128, 128)
    bspec = pl.BlockSpec((1, N // 128, 128), lambda b: (b, 0, 0))
    mspec = pl.BlockSpec((1, 8, 128), lambda b: (b, 0, 0))
    skh, skl, meta = pl.pallas_call(
        _key_kernel,
        grid=(B,),
        in_specs=[bspec, bspec, bspec, bspec],
        out_specs=[bspec, bspec, mspec],
        out_shape=[
            jax.ShapeDtypeStruct((B, N // 128, 128), jnp.int32),
            jax.ShapeDtypeStruct((B, N // 128, 128), jnp.int32),
            jax.ShapeDtypeStruct((B, 8, 128), jnp.int32),
        ],
    )(r(q0t), r(q1t), r(q2t), r(padt))
    return skh.reshape(B, N), skl.reshape(B, N), meta.reshape(B, 8 * 128)


def _make_sc_kernel(B, N, C):
    NI = N // 16
    CH = 128
    mesh = plsc.VectorSubcoreMesh(core_axis_name="c", subcore_axis_name="s")
    LANE = lambda: lax.iota(jnp.int32, 16)

    @functools.partial(
        pl.kernel,
        out_type=(
            jax.ShapeDtypeStruct((B, N), jnp.int32),
            jax.ShapeDtypeStruct((B, N), jnp.int32),
            jax.ShapeDtypeStruct((B, N, C), jnp.float32),
        ),
        mesh=mesh,
        scratch_types=[
            pltpu.VMEM((N,), jnp.int32),
            pltpu.VMEM((N,), jnp.int32),
            pltpu.VMEM((N,), jnp.int32),
            pltpu.VMEM((N,), jnp.int32),
            pltpu.VMEM((N,), jnp.int32),
            pltpu.VMEM((N,), jnp.int32),
            pltpu.VMEM((4096,), jnp.int32),
            pltpu.VMEM((16,), jnp.int32),
            pltpu.VMEM((128,), jnp.int32),
            pltpu.VMEM((128, 128), jnp.float32),
            pltpu.SemaphoreType.DMA,
        ],
    )
    def sc_kernel(skh_hbm, skl_hbm, meta_hbm, unified_hbm,
                  idx_out, pad_out, tok_out,
                  Ah, Al, Av, Bh, Bl, Bv, hist, meta_v, idxg, rows, sem):
        c = lax.axis_index("c")
        s = lax.axis_index("s")

        @pl.when(s < 4)
        def _sort():
            b = 2 * s + c
            pltpu.sync_copy(skh_hbm.at[b], Ah)
            pltpu.sync_copy(skl_hbm.at[b], Al)
            pltpu.sync_copy(meta_hbm.at[b, pl.ds(0, 16)], meta_v)
            sph = jnp.max(meta_v[...], axis=0)
            pltpu.sync_copy(meta_hbm.at[b, pl.ds(128, 16)], meta_v)
            spl = jnp.max(meta_v[...], axis=0)
            pltpu.sync_copy(meta_hbm.at[b, pl.ds(256, 16)], meta_v)
            npass = jnp.max(meta_v[...], axis=0)
            np2 = npass + (npass & 1)

            def one_pass(p, srcH, srcL, srcV, dstH, dstL, dstV):
                shift = 8 * (p % 4)
                use_hi = p >= 4
                is_last = jnp.int32(p) == np2 - 1

                def dig(kh_v, kl_v):
                    x = kh_v if use_hi else kl_v
                    return lax.shift_right_logical(x, shift) & jnp.int32(0xFF)

                def zero_body(j, carry):
                    hist[pl.ds(j * 16, 16)] = jnp.zeros((16,), jnp.int32)
                    return carry
                lax.fori_loop(0, 256, zero_body, 0)

                def hist_body(i, carry):
                    kh_v = srcH[pl.ds(i * 16, 16)]
                    kl_v = srcL[pl.ds(i * 16, 16)]
                    idx = dig(kh_v, kl_v) * 16 + LANE()
                    g = plsc.load_gather(hist, [idx])
                    plsc.store_scatter(hist, [idx], g + 1)
                    return carry
                lax.fori_loop(0, NI, hist_body, 0)

                def scan_body(j, carry):
                    v = hist[pl.ds(j * 16, 16)]
                    cum = plsc.cumsum(v)
                    hist[pl.ds(j * 16, 16)] = cum - v + carry
                    return carry + jnp.max(cum, axis=0)
                lax.fori_loop(0, 256, scan_body, jnp.int32(0))

                def perm_body(i, carry):
                    kh_v = srcH[pl.ds(i * 16, 16)]
                    kl_v = srcL[pl.ds(i * 16, 16)]
                    if p == 0:
                        v_v = LANE() * jnp.int32(NI) + i.astype(jnp.int32)
                    else:
                        v_v = srcV[pl.ds(i * 16, 16)]
                    idx = dig(kh_v, kl_v) * 16 + LANE()
                    r = plsc.load_gather(hist, [idx])
                    plsc.store_scatter(hist, [idx], r + 1)
                    pos_t = (r & jnp.int32(NI - 1)) * 16 \
                        + lax.shift_right_logical(r, 9)
                    pos = jnp.where(is_last, r, pos_t)
                    plsc.store_scatter(dstH, [pos], kh_v)
                    plsc.store_scatter(dstL, [pos], kl_v)
                    plsc.store_scatter(dstV, [pos], v_v)
                    return carry
                lax.fori_loop(0, NI, perm_body, 0)

            for p in range(8):
                srcH, srcL, srcV = (Ah, Al, Av) if p % 2 == 0 else (Bh, Bl, Bv)
                dstH, dstL, dstV = (Bh, Bl, Bv) if p % 2 == 0 else (Ah, Al, Av)
                if p == 0:
                    one_pass(p, srcH, srcL, srcV, dstH, dstL, dstV)
                else:
                    @pl.when(jnp.int32(p) < np2)
                    def _run(p=p, sH=srcH, sL=srcL, sV=srcV,
                             dH=dstH, dL=dstL, dV=dstV):
                        one_pass(p, sH, sL, sV, dH, dL, dV)

            pltpu.sync_copy(Av, idx_out.at[b])

            def pad_body(i, carry):
                sh_v = Ah[pl.ds(i * 16, 16)]
                sl_v = Al[pl.ds(i * 16, 16)]
                Bv[pl.ds(i * 16, 16)] = \
                    ((sh_v == sph) & (sl_v == spl)).astype(jnp.int32)
                return carry
            lax.fori_loop(0, NI, pad_body, 0)
            pltpu.sync_copy(Bv, pad_out.at[b])

        plsc.subcore_barrier()

        bb = 2 * (s // 4) + c
        rowbase = (s % 4) * (N // 4)

        def gather_body(k, carry):
            start = rowbase + k * CH
            pltpu.sync_copy(idx_out.at[bb, pl.ds(start, CH)], idxg)
            pltpu.async_copy(unified_hbm.at[bb].at[idxg], rows, sem).wait()
            pltpu.sync_copy(rows, tok_out.at[bb, pl.ds(start, CH)])
            return carry
        lax.fori_loop(0, (N // 4) // CH, gather_body, 0)

    return sc_kernel


def kernel(lidar_tokens, lidar_coords, img_tokens, img_coords, K, T_c2w,
           lidar_padding_mask, img_padding_mask):
    B, N_lidar, C = lidar_tokens.shape
    N_img = img_tokens.shape[1]
    N = N_lidar + N_img
    num_views = K.shape[1]
    view_id = min(DEFAULT_MAIN_VIEW, num_views - 1)

    view_indices = jnp.full((B,), view_id, dtype=jnp.int64)
    bidx = jnp.arange(B)
    sel_K = K[bidx, view_indices]
    sel_T = T_c2w[bidx, view_indices]
    xyz1 = jnp.concatenate(
        [lidar_coords, jnp.ones_like(lidar_coords[..., :1])], axis=-1)
    invT = jnp.linalg.inv(sel_T)
    cam_homo = jnp.einsum('bij,bnj->bni', invT, xyz1)
    img_homo = jnp.einsum('bij,bnj->bni', sel_K, cam_homo[..., :3])
    depth = img_homo[..., 2:3]
    uv = img_homo[..., :2] / jnp.clip(depth, 1e-05, None)
    camera_ids = jnp.broadcast_to(
        view_indices.reshape(B, 1, 1).astype(uv.dtype), (B, N_lidar, 1))
    projected = jnp.concatenate([camera_ids, uv], axis=-1)
    valid = depth[..., 0] > 1e-05
    lidar_pad = lidar_padding_mask | (~valid)
    cam_ids_img = img_coords[..., 0].astype(jnp.int64)
    pad_full = jnp.concatenate(
        [lidar_pad, img_padding_mask | (cam_ids_img != view_id)], axis=1)
    coords_full = jnp.concatenate([projected, img_coords], axis=1)
    qf = jnp.floor(coords_full / GRID)

    unified_tokens = jnp.concatenate([lidar_tokens, img_tokens], axis=1)

    tp = lambda x: x.reshape(B, 16, N // 16).transpose(0, 2, 1).reshape(B, N)
    q0t = tp(qf[..., 0].astype(jnp.float32))
    q1t = tp(qf[..., 1].astype(jnp.float32))
    q2t = tp(qf[..., 2].astype(jnp.float32))
    padt = tp(pad_full.astype(jnp.int32))

    skh, skl, meta = _compute_keys_tc(q0t, q1t, q2t, padt)

    sc = _make_sc_kernel(B, N, C)
    idx_i32, pad_i32, tokens = sc(skh, skl, meta, unified_tokens)

    sorted_indices = idx_i32.astype(jnp.int64)
    sorted_pad = pad_i32.astype(jnp.bool_)
    return (tokens, sorted_indices, sorted_pad, N_lidar)


# speedup vs baseline: 1.5728x; 1.5728x over previous
import functools

import jax
import jax.numpy as jnp
import numpy as np
from jax import lax
from jax.experimental import pallas as pl
from jax.experimental.pallas import tpu as pltpu
from jax.experimental.pallas import tpu_sc as plsc

GRID = 0.001
DEFAULT_MAIN_VIEW = 0

_SIGN = np.int32(-2**31)
_C1 = 1
_C0 = 34464


def _ult(a, b):
    return (a ^ _SIGN) < (b ^ _SIGN)


def _uge_c(a, c):
    return ((a ^ _SIGN) >= (jnp.int32(c) ^ _SIGN)).astype(jnp.int32)


def _add64(ah, al, bh, bl):
    lo = al + bl
    carry = _ult(lo, al).astype(jnp.int32)
    return ah + bh + carry, lo


def _sub64(ah, al, bh, bl):
    borrow = _ult(al, bl).astype(jnp.int32)
    return ah - bh - borrow, al - bl


def _mul64_100000(h, l):
    l1 = lax.shift_right_logical(l, jnp.int32(16))
    l0 = l & jnp.int32(0xFFFF)
    plo = l0 * jnp.int32(_C0)
    mid = l0 * jnp.int32(_C1) + l1 * jnp.int32(_C0)
    lo = plo + lax.shift_left(mid & jnp.int32(0xFFFF), jnp.int32(16))
    carry = _ult(lo, plo).astype(jnp.int32)
    hi = lax.shift_right_logical(mid, jnp.int32(16)) + l1 * jnp.int32(_C1) + carry \
        + h * jnp.int32(100000)
    return hi, lo


def _split64(qf):
    neg = qf < 0.0
    a = jnp.abs(qf)
    hif = jnp.floor(a * jnp.float32(2.0**-32))
    ha = hif.astype(jnp.int32)
    remf = a - hif * jnp.float32(2.0**32)
    big = remf >= jnp.float32(2.0**31)
    lo_small = jnp.where(big, jnp.float32(0.0), remf).astype(jnp.int32)
    lo_big = jnp.where(big, remf - jnp.float32(2.0**31),
                       jnp.float32(0.0)).astype(jnp.int32) | _SIGN
    la = jnp.where(big, lo_big, lo_small)
    nlo = -la
    nhi = -ha - jnp.where(la == jnp.int32(0), jnp.int32(0), jnp.int32(1))
    return jnp.where(neg, nhi, ha), jnp.where(neg, nlo, la)


def _masked_min64(hi, lo, mask):
    mh = jnp.min(jnp.where(mask, hi, jnp.int32(2**31 - 1)))
    sel = mask & (hi == mh)
    ml = jnp.min(jnp.where(sel, lo ^ _SIGN, jnp.int32(2**31 - 1))) ^ _SIGN
    return mh, ml


def _masked_max64(hi, lo, mask):
    mh = jnp.max(jnp.where(mask, hi, _SIGN))
    sel = mask & (hi == mh)
    ml = jnp.max(jnp.where(sel, lo ^ _SIGN, _SIGN)) ^ _SIGN
    return mh, ml


def _key_kernel(q0_ref, q1_ref, q2_ref, pad_ref, skh_ref, skl_ref, meta_ref):
    q0f = q0_ref[0]
    q1f = q1_ref[0]
    q2f = q2_ref[0]
    padm = pad_ref[0] != 0
    valid = ~padm
    any_valid = jnp.max(valid.astype(jnp.int32)) > 0

    q0 = q0f.astype(jnp.int32)
    q1h, q1l = _split64(q1f)
    q2h, q2l = _split64(q2f)

    m0 = jnp.min(jnp.where(valid, q0, jnp.int32(2**31 - 1)))
    m1h, m1l = _masked_min64(q1h, q1l, valid)
    m2h, m2l = _masked_min64(q2h, q2l, valid)
    zero = jnp.int32(0)
    m0 = jnp.where(any_valid, m0, zero)
    m1h = jnp.where(any_valid, m1h, zero)
    m1l = jnp.where(any_valid, m1l, zero)
    m2h = jnp.where(any_valid, m2h, zero)
    m2l = jnp.where(any_valid, m2l, zero)

    d0 = q0 - m0
    d1h, d1l = _sub64(q1h, q1l, m1h, m1l)
    d2h, d2l = _sub64(q2h, q2l, m2h, m2l)

    x1h, x1l = _masked_max64(d1h, d1l, valid)
    x2h, x2l = _masked_max64(d2h, d2l, valid)
    x1h = jnp.where(any_valid, x1h, zero)
    x1l = jnp.where(any_valid, x1l, zero)
    x2h = jnp.where(any_valid, x2h, zero)
    x2l = jnp.where(any_valid, x2l, zero)

    p0 = (d0 & 1) == 1
    p01 = ((d0 ^ d1l) & 1) == 1
    a1h, a1l = _sub64(x1h, x1l, d1h, d1l)
    s1h = jnp.where(p0, a1h, d1h)
    s1l = jnp.where(p0, a1l, d1l)
    a2h, a2l = _sub64(x2h, x2l, d2h, d2l)
    s2h = jnp.where(p01, a2h, d2h)
    s2l = jnp.where(p01, a2l, d2l)

    k1h, k1l = _add64(jnp.zeros_like(d0), d0 * jnp.int32(100000), s1h, s1l)
    k2h, k2l = _mul64_100000(k1h, k1l)
    kh, kl = _add64(k2h, k2l, s2h, s2l)

    vh, vl = _masked_max64(kh, kl, valid)
    ph, plo = _add64(vh, vl, zero, jnp.int32(1))
    kh = jnp.where(padm, ph, kh)
    kl = jnp.where(padm, plo, kl)

    ukh = kh ^ _SIGN
    mnh_b = jnp.min(kh)
    mnl = jnp.min(jnp.where(kh == mnh_b, kl ^ _SIGN, jnp.int32(2**31 - 1))) ^ _SIGN
    mxh_b = jnp.max(kh)
    mxl = jnp.max(jnp.where(kh == mxh_b, kl ^ _SIGN, _SIGN)) ^ _SIGN
    umnh = mnh_b ^ _SIGN
    umxh = mxh_b ^ _SIGN
    skh, skl = _sub64(ukh, kl, umnh, mnl)
    sph, spl = _sub64(ph ^ _SIGN, plo, umnh, mnl)
    rh, rl = _sub64(umxh, mxl, umnh, mnl)
    cnt_lo = jnp.int32(1) + _uge_c(rl, 1 << 8) + _uge_c(rl, 1 << 16) \
        + _uge_c(rl, 1 << 24)
    cnt_hi = jnp.int32(5) + _uge_c(rh, 1 << 8) + _uge_c(rh, 1 << 16) \
        + _uge_c(rh, 1 << 24)
    npass = jnp.where(rh == 0, cnt_lo, cnt_hi)

    skh_ref[0] = skh
    skl_ref[0] = skl
    meta_ref[0, 0:1, :] = jnp.full((1, 128), sph, jnp.int32)
    meta_ref[0, 1:2, :] = jnp.full((1, 128), spl, jnp.int32)
    meta_ref[0, 2:3, :] = jnp.full((1, 128), npass, jnp.int32)


def _compute_keys_tc(q0t, q1t, q2t, padt):
    B, N = q0t.shape
    r = lambda x: x.reshape(B, N // 128, 128)
    bspec = pl.BlockSpec((1, N // 128, 128), lambda b: (b, b * 0, b * 0))
    mspec = pl.BlockSpec((1, 8, 128), lambda b: (b, b * 0, b * 0))
    skh, skl, meta = pl.pallas_call(
        _key_kernel,
        grid=(B,),
        in_specs=[bspec, bspec, bspec, bspec],
        out_specs=[bspec, bspec, mspec],
        out_shape=[
            jax.ShapeDtypeStruct((B, N // 128, 128), jnp.int32),
            jax.ShapeDtypeStruct((B, N // 128, 128), jnp.int32),
            jax.ShapeDtypeStruct((B, 8, 128), jnp.int32),
        ],
    )(r(q0t), r(q1t), r(q2t), r(padt))
    return skh.reshape(B, N), skl.reshape(B, N), meta.reshape(B, 8 * 128)


def _make_sc_kernel(B, N, C):
    NI = N // 16
    CH = 128
    mesh = plsc.VectorSubcoreMesh(core_axis_name="c", subcore_axis_name="s")
    LANE = lambda: lax.iota(jnp.int32, 16)

    @functools.partial(
        pl.kernel,
        out_type=(
            jax.ShapeDtypeStruct((B, N), jnp.int32),
            jax.ShapeDtypeStruct((B, N), jnp.int32),
            jax.ShapeDtypeStruct((B, N, C), jnp.float32),
        ),
        mesh=mesh,
        compiler_params=pltpu.CompilerParams(needs_layout_passes=False),
        scratch_types=[
            pltpu.VMEM((N,), jnp.int32),
            pltpu.VMEM((N,), jnp.int32),
            pltpu.VMEM((N,), jnp.int32),
            pltpu.VMEM((N,), jnp.int32),
            pltpu.VMEM((N,), jnp.int32),
            pltpu.VMEM((N,), jnp.int32),
            pltpu.VMEM((4096,), jnp.int32),
            pltpu.VMEM((16,), jnp.int32),
            pltpu.VMEM((128,), jnp.int32),
            pltpu.VMEM((128, 128), jnp.float32),
            pltpu.SemaphoreType.DMA,
        ],
    )
    def sc_kernel(skh_hbm, skl_hbm, meta_hbm, unified_hbm,
                  idx_out, pad_out, tok_out,
                  Ah, Al, Av, Bh, Bl, Bv, hist, meta_v, idxg, rows, sem):
        c = lax.axis_index("c")
        s = lax.axis_index("s")

        @pl.when(s < jnp.int32(4))
        def _sort():
            b = jnp.int32(2) * s + c
            pltpu.sync_copy(skh_hbm.at[b], Ah)
            pltpu.sync_copy(skl_hbm.at[b], Al)
            pltpu.sync_copy(meta_hbm.at[b, pl.ds(0, 16)], meta_v)
            sph = meta_v[...]
            pltpu.sync_copy(meta_hbm.at[b, pl.ds(128, 16)], meta_v)
            spl = meta_v[...]
            pltpu.sync_copy(meta_hbm.at[b, pl.ds(256, 16)], meta_v)
            npass = meta_v[...][0]
            np2 = npass + (npass & jnp.int32(1))
            np2_v = meta_v[...] + (meta_v[...] & jnp.int32(1))

            def one_pass(p, srcH, srcL, srcV, dstH, dstL, dstV, np2_v):
                shift = 8 * (p % 4)
                use_hi = p >= 4
                is_last = jnp.full((16,), p, jnp.int32) == np2_v - jnp.int32(1)

                def dig(kh_v, kl_v):
                    x = kh_v if use_hi else kl_v
                    return lax.shift_right_logical(x, jnp.int32(shift)) & jnp.int32(0xFF)

                def zero_body(j, carry):
                    hist[pl.ds(j * jnp.int32(16), 16)] = jnp.zeros((16,), jnp.int32)
                    return carry
                lax.fori_loop(jnp.int32(0), jnp.int32(256), zero_body, jnp.int32(0))

                def hist_body(i, carry):
                    kh_v = srcH[pl.ds(i * jnp.int32(16), 16)]
                    kl_v = srcL[pl.ds(i * jnp.int32(16), 16)]
                    idx = dig(kh_v, kl_v) * jnp.int32(16) + LANE()
                    g = plsc.load_gather(hist, [idx])
                    plsc.store_scatter(hist, [idx], g + 1)
                    return carry
                lax.fori_loop(jnp.int32(0), jnp.int32(NI), hist_body, jnp.int32(0))

                def scan_body(j, carry):
                    v = hist[pl.ds(j * jnp.int32(16), 16)]
                    cum = plsc.cumsum(v)
                    hist[pl.ds(j * jnp.int32(16), 16)] = cum - v + carry
                    meta_v[...] = cum
                    last = jnp.full((16,), 15, jnp.int32)
                    return carry + plsc.load_gather(meta_v, [last])
                lax.fori_loop(jnp.int32(0), jnp.int32(256), scan_body,
                              jnp.zeros((16,), jnp.int32))

                def perm_body(i, carry):
                    kh_v = srcH[pl.ds(i * jnp.int32(16), 16)]
                    kl_v = srcL[pl.ds(i * jnp.int32(16), 16)]
                    if p == 0:
                        v_v = LANE() * jnp.int32(NI) + jnp.int32(1) * i.astype(jnp.int32)
                    else:
                        v_v = srcV[pl.ds(i * jnp.int32(16), 16)]
                    idx = dig(kh_v, kl_v) * jnp.int32(16) + LANE()
                    r = plsc.load_gather(hist, [idx])
                    plsc.store_scatter(hist, [idx], r + 1)
                    pos_t = (r & jnp.int32(NI - 1)) * jnp.int32(16) \
                        + lax.shift_right_logical(r, jnp.int32(9))
                    pos = jnp.where(is_last, r, pos_t)
                    plsc.store_scatter(dstH, [pos], kh_v)
                    plsc.store_scatter(dstL, [pos], kl_v)
                    plsc.store_scatter(dstV, [pos], v_v)
                    return carry
                lax.fori_loop(jnp.int32(0), jnp.int32(NI), perm_body, jnp.int32(0))

            for p in range(8):
                srcH, srcL, srcV = (Ah, Al, Av) if p % 2 == 0 else (Bh, Bl, Bv)
                dstH, dstL, dstV = (Bh, Bl, Bv) if p % 2 == 0 else (Ah, Al, Av)
                if p == 0:
                    one_pass(p, srcH, srcL, srcV, dstH, dstL, dstV, np2_v)
                else:
                    @pl.when(jnp.int32(p) < np2)
                    def _run(p=p, sH=srcH, sL=srcL, sV=srcV,
                             dH=dstH, dL=dstL, dV=dstV):
                        one_pass(p, sH, sL, sV, dH, dL, dV, np2_v)

            pltpu.sync_copy(Av, idx_out.at[b])

            def pad_body(i, carry):
                sh_v = Ah[pl.ds(i * jnp.int32(16), 16)]
                sl_v = Al[pl.ds(i * jnp.int32(16), 16)]
                Bv[pl.ds(i * jnp.int32(16), 16)] = \
                    ((sh_v == sph) & (sl_v == spl)).astype(jnp.int32)
                return carry
            lax.fori_loop(jnp.int32(0), jnp.int32(NI), pad_body, jnp.int32(0))
            pltpu.sync_copy(Bv, pad_out.at[b])

        plsc.subcore_barrier()

        bb = jnp.int32(2) * (s // jnp.int32(4)) + c
        rowbase = (s % jnp.int32(4)) * jnp.int32(N // 4)

        def gather_body(k, carry):
            start = rowbase + k * jnp.int32(CH)
            pltpu.sync_copy(idx_out.at[bb, pl.ds(start, CH)], idxg)
            pltpu.async_copy(unified_hbm.at[bb].at[idxg], rows, sem).wait()
            pltpu.sync_copy(rows, tok_out.at[bb, pl.ds(start, CH)])
            return carry
        lax.fori_loop(jnp.int32(0), jnp.int32((N // 4) // CH), gather_body, jnp.int32(0))

    return sc_kernel


def kernel(lidar_tokens, lidar_coords, img_tokens, img_coords, K, T_c2w,
           lidar_padding_mask, img_padding_mask):
    B, N_lidar, C = lidar_tokens.shape
    N_img = img_tokens.shape[1]
    N = N_lidar + N_img
    num_views = K.shape[1]
    view_id = min(DEFAULT_MAIN_VIEW, num_views - 1)

    view_indices = jnp.full((B,), view_id, dtype=jnp.int64)
    bidx = jnp.arange(B)
    sel_K = K[bidx, view_indices]
    sel_T = T_c2w[bidx, view_indices]
    xyz1 = jnp.concatenate(
        [lidar_coords, jnp.ones_like(lidar_coords[..., :1])], axis=-1)
    invT = jnp.linalg.inv(sel_T)
    cam_homo = jnp.einsum('bij,bnj->bni', invT, xyz1)
    img_homo = jnp.einsum('bij,bnj->bni', sel_K, cam_homo[..., :3])
    depth = img_homo[..., 2:3]
    uv = img_homo[..., :2] / jnp.clip(depth, 1e-05, None)
    camera_ids = jnp.broadcast_to(
        view_indices.reshape(B, 1, 1).astype(uv.dtype), (B, N_lidar, 1))
    projected = jnp.concatenate([camera_ids, uv], axis=-1)
    valid = depth[..., 0] > 1e-05
    lidar_pad = lidar_padding_mask | (~valid)
    cam_ids_img = img_coords[..., 0].astype(jnp.int64)
    pad_full = jnp.concatenate(
        [lidar_pad, img_padding_mask | (cam_ids_img != view_id)], axis=1)
    coords_full = jnp.concatenate([projected, img_coords], axis=1)
    qf = jnp.floor(coords_full / GRID)

    unified_tokens = jnp.concatenate([lidar_tokens, img_tokens], axis=1)

    tp = lambda x: x.reshape(B, 16, N // 16).transpose(0, 2, 1).reshape(B, N)
    q0t = tp(qf[..., 0].astype(jnp.float32))
    q1t = tp(qf[..., 1].astype(jnp.float32))
    q2t = tp(qf[..., 2].astype(jnp.float32))
    padt = tp(pad_full.astype(jnp.int32))

    skh, skl, meta = _compute_keys_tc(q0t, q1t, q2t, padt)

    sc = _make_sc_kernel(B, N, C)
    idx_i32, pad_i32, tokens = sc(skh, skl, meta, unified_tokens)

    sorted_indices = idx_i32.astype(jnp.int64)
    sorted_pad = pad_i32.astype(jnp.bool_)
    return (tokens, sorted_indices, sorted_pad, N_lidar)
